# baseline (speedup 1.0000x reference)
import functools

import jax
import jax.numpy as jnp
from jax import lax
from jax.experimental import pallas as pl
from jax.experimental.pallas import tpu as pltpu
from jax.experimental.pallas import tpu_sc as plsc

NUM_ROWS = 1000000
DIM = 64
N_EX = 16384
N_TOK = 200

N_WORKERS = 32
EX_PER_W = N_EX // N_WORKERS
EX_CHUNK = 128
N_BC = EX_PER_W // EX_CHUNK
TV_STRIDE = EX_CHUNK + 5
NG = 4


def _make_kernel():
    mesh = plsc.VectorSubcoreMesh(core_axis_name="c", subcore_axis_name="s")
    nc = plsc.get_sparse_core_info().num_cores

    @functools.partial(
        pl.kernel,
        mesh=mesh,
        out_type=jax.ShapeDtypeStruct((N_TOK, DIM, N_EX), jnp.float32),
        scratch_types=[
            pltpu.VMEM((EX_CHUNK * N_TOK,), jnp.int32),
            [pltpu.VMEM((EX_CHUNK,), jnp.int32) for _ in range(NG)],
            [pltpu.VMEM((EX_CHUNK, 2 * DIM), jnp.float32) for _ in range(NG)],
            [pltpu.VMEM((DIM, TV_STRIDE), jnp.float32) for _ in range(2)],
            [pltpu.SemaphoreType.DMA for _ in range(NG)],
            [pltpu.SemaphoreType.DMA for _ in range(2)],
        ],
        compiler_params=pltpu.CompilerParams(needs_layout_passes=False),
    )
    def emb_kernel(idx_hbm, table_hbm, out_hbm, idx_v, lv, gv, tv, sg, so):
        wid = lax.axis_index("s") * nc + lax.axis_index("c")
        iota = lax.iota(jnp.int32, 16)
        iota_tok = iota * N_TOK
        zero16 = iota * 0

        def bc_body(bc, _):
            ex0 = wid * EX_PER_W + bc * EX_CHUNK
            pltpu.sync_copy(
                idx_hbm.at[pl.ds(pl.multiple_of(ex0 * N_TOK, 8), EX_CHUNK * N_TOK)],
                idx_v,
            )

            def prep_and_gather(t, j):
                for b16 in range(EX_CHUNK // 16):
                    v = plsc.load_gather(idx_v, [iota_tok + (b16 * 16 * N_TOK + t)])
                    lv[j][pl.ds(b16 * 16, 16)] = v
                pltpu.make_async_copy(table_hbm.at[lv[j]], gv[j], sg[j]).start()

            def transpose(j, p):
                @plsc.parallel_loop(0, EX_CHUNK, unroll=16)
                def _(r):
                    rv = zero16 + r
                    for q in range(DIM // 16):
                        plsc.store_scatter(
                            tv[p], [iota + q * 16, rv], gv[j][r, pl.ds(q * 16, 16)]
                        )

            def store_copy(t, p):
                return pltpu.make_async_copy(
                    tv[p].at[:, pl.ds(0, EX_CHUNK)],
                    out_hbm.at[t, :, pl.ds(pl.multiple_of(ex0, 128), EX_CHUNK)],
                    so[p],
                )

            def gather_wait(j):
                pltpu.make_async_copy(table_hbm.at[lv[j]], gv[j], sg[j]).wait()

            for j in range(NG):
                prep_and_gather(j, j)

            def tc_body(t, _):
                for j in range(NG):
                    g = NG * t + j
                    p = j % 2
                    gather_wait(j)
                    if j < 2:
                        @pl.when(t > 0)
                        def _():
                            store_copy(g - 2, p).wait()
                    else:
                        store_copy(g - 2, p).wait()
                    transpose(j, p)
                    store_copy(g, p).start()

                    @pl.when(t < N_TOK // NG - 1)
                    def _():
                        prep_and_gather(g + NG, j)

                return 0

            lax.fori_loop(0, N_TOK // NG, tc_body, 0)
            store_copy(N_TOK - 2, 0).wait()
            store_copy(N_TOK - 1, 1).wait()
            return 0

        lax.fori_loop(0, N_BC, bc_body, 0)

    return emb_kernel


def kernel(indices, weight):
    idx_flat = indices.reshape(-1).astype(jnp.int32)
    table_pad = jnp.pad(weight, ((0, 0), (0, DIM)))
    out_phys = _make_kernel()(idx_flat, table_pad)
    return jnp.transpose(out_phys, (2, 0, 1))


# speedup vs baseline: 2.8644x; 2.8644x over previous
import functools

import jax
import jax.numpy as jnp
from jax import lax
from jax.experimental import pallas as pl
from jax.experimental.pallas import tpu as pltpu
from jax.experimental.pallas import tpu_sc as plsc

NUM_ROWS = 1000000
DIM = 64
N_EX = 16384
N_TOK = 200

N_WORKERS = 32
EX_PER_W = N_EX // N_WORKERS
EX_CHUNK = 128
N_BC = EX_PER_W // EX_CHUNK
TV_STRIDE = EX_CHUNK + 5
NG = 4


def _make_kernel():
    mesh = plsc.VectorSubcoreMesh(core_axis_name="c", subcore_axis_name="s")
    nc = plsc.get_sparse_core_info().num_cores

    @functools.partial(
        pl.kernel,
        mesh=mesh,
        out_type=jax.ShapeDtypeStruct((N_TOK, 8, N_EX // 128, 8, 128), jnp.float32),
        scratch_types=[
            pltpu.VMEM((EX_CHUNK * N_TOK,), jnp.int32),
            [pltpu.VMEM((EX_CHUNK,), jnp.int32) for _ in range(NG)],
            [pltpu.VMEM((EX_CHUNK, DIM), jnp.float32) for _ in range(NG)],
            [pltpu.VMEM((8, 8, TV_STRIDE), jnp.float32) for _ in range(2)],
            [pltpu.SemaphoreType.DMA for _ in range(NG)],
            [pltpu.SemaphoreType.DMA for _ in range(2)],
        ],
        compiler_params=pltpu.CompilerParams(
            needs_layout_passes=False, use_tc_tiling_on_sc=False
        ),
    )
    def emb_kernel(idx_hbm, table_hbm, out_hbm, idx_v, lv, gv, tv, sg, so):
        wid = lax.axis_index("s") * nc + lax.axis_index("c")
        iota = lax.iota(jnp.int32, 16)
        iota_tok = iota * N_TOK
        zero16 = iota * 0

        def bc_body(bc, _):
            ex0 = wid * EX_PER_W + bc * EX_CHUNK
            del ex0
            pltpu.sync_copy(
                idx_hbm.at[
                    pl.ds(
                        pl.multiple_of(
                            (wid * EX_PER_W + bc * EX_CHUNK) * N_TOK, 8
                        ),
                        EX_CHUNK * N_TOK,
                    )
                ],
                idx_v,
            )

            def prep_and_gather(t, j):
                for b16 in range(EX_CHUNK // 16):
                    v = plsc.load_gather(idx_v, [iota_tok + (b16 * 16 * N_TOK + t)])
                    lv[j][pl.ds(b16 * 16, 16)] = v
                pltpu.make_async_copy(table_hbm.at[lv[j]], gv[j], sg[j]).start()

            def transpose(j, p):
                @plsc.parallel_loop(0, EX_CHUNK, unroll=16)
                def _(r):
                    rv = zero16 + r
                    for q in range(DIM // 16):
                        dvec = iota + q * 16
                        plsc.store_scatter(
                            tv[p],
                            [dvec // 8, dvec % 8, rv],
                            gv[j][r, pl.ds(q * 16, 16)],
                        )

            def store_copy(t, p):
                bcol = wid * N_BC + bc
                return pltpu.make_async_copy(
                    tv[p].at[:, :, pl.ds(0, EX_CHUNK)],
                    out_hbm.at[t, :, bcol, :, :],
                    so[p],
                )

            def gather_wait(j):
                pltpu.make_async_copy(table_hbm.at[lv[j]], gv[j], sg[j]).wait()

            for j in range(NG):
                prep_and_gather(j, j)

            def tc_body(t, _):
                for j in range(NG):
                    g = NG * t + j
                    p = j % 2
                    gather_wait(j)
                    if j < 2:
                        @pl.when(t > 0)
                        def _():
                            store_copy(g - 2, p).wait()
                    else:
                        store_copy(g - 2, p).wait()
                    transpose(j, p)
                    store_copy(g, p).start()

                    @pl.when(t < N_TOK // NG - 1)
                    def _():
                        prep_and_gather(g + NG, j)

                return 0

            lax.fori_loop(0, N_TOK // NG, tc_body, 0)
            store_copy(N_TOK - 2, 0).wait()
            store_copy(N_TOK - 1, 1).wait()
            return 0

        lax.fori_loop(0, N_BC, bc_body, 0)

    return emb_kernel


def kernel(indices, weight):
    idx_flat = indices.reshape(-1).astype(jnp.int32)
    out5 = _make_kernel()(idx_flat, weight)
    return jnp.transpose(out5, (2, 4, 0, 1, 3)).reshape(N_EX, N_TOK, DIM)
